# baseline (speedup 1.0000x reference)
import jax
import jax.numpy as jnp
from jax import lax
from jax.experimental import pallas as pl
from jax.experimental.pallas import tpu as pltpu
from jax.experimental.pallas import tpu_sc as plsc

B = 16384
D = 32
WD = 1e-05
NC = 2
NS = 16
NW = NC * NS
BPW = B // NW
L = 16
V = 1000000


def _bpr_body(u_hbm, i_hbm, j_hbm, w1_hbm, h1_hbm, out_hbm,
              u_v, i_v, j_v, uix, iix, jix, ub, ib, jb, out_v, sem):
    wid = lax.axis_index("s") * NC + lax.axis_index("c")
    base = wid * BPW

    pltpu.sync_copy(u_hbm.at[pl.ds(base, BPW)], u_v)
    pltpu.sync_copy(i_hbm.at[pl.ds(base, BPW)], i_v)
    pltpu.sync_copy(j_hbm.at[pl.ds(base, BPW)], j_v)

    def build(d, _):
        def g_body(g, _2):
            sl = pl.ds(g * L, L)
            dsl = pl.ds(d * BPW + g * L, L)
            off = d * V
            uix[dsl] = u_v[sl] + off
            iix[dsl] = i_v[sl] + off
            jix[dsl] = j_v[sl] + off
            return _2
        lax.fori_loop(0, BPW // L, g_body, None)
        return _

    lax.fori_loop(0, D, build, None)

    for d in range(D):
        dsl = pl.ds(d * BPW, BPW)
        pltpu.async_copy(w1_hbm.at[uix.at[dsl]], ub.at[dsl], sem)
        pltpu.async_copy(h1_hbm.at[iix.at[dsl]], ib.at[dsl], sem)
        pltpu.async_copy(h1_hbm.at[jix.at[dsl]], jb.at[dsl], sem)
    for d in range(D):
        dsl = pl.ds(d * BPW, BPW)
        pltpu.make_async_copy(w1_hbm.at[uix.at[dsl]], ub.at[dsl], sem).wait()
        pltpu.make_async_copy(h1_hbm.at[iix.at[dsl]], ib.at[dsl], sem).wait()
        pltpu.make_async_copy(h1_hbm.at[jix.at[dsl]], jb.at[dsl], sem).wait()

    third = jnp.float32(1.0 / 3.0)
    fifth = jnp.float32(1.0 / 5.0)
    seventh = jnp.float32(1.0 / 7.0)
    zero = jnp.zeros((L,), jnp.float32)

    def g_compute(g, _):
        g0 = g * L

        def d_body(d, carry):
            accg, accr = carry
            dsl = pl.ds(d * BPW + g0, L)
            ue = ub[dsl]
            ie = ib[dsl]
            je = jb[dsl]
            x = ue * (ie - je)
            e = jnp.exp(-jnp.abs(x))
            s = e / (e + 2.0)
            s2 = s * s
            log1p = (2.0 * s) * (1.0 + s2 * (third + s2 * (fifth + s2 * seventh)))
            accg = accg + (jnp.minimum(x, zero) - log1p)
            accr = accr + (ue * ue + ie * ie + je * je)
            return accg, accr

        accg, accr = lax.fori_loop(0, D, d_body, (zero, zero))
        out_v[pl.ds(g0, L)] = (jnp.float32(WD / D) * accr
                               - jnp.float32(1.0 / D) * accg)
        return _

    lax.fori_loop(0, BPW // L, g_compute, None)
    pltpu.sync_copy(out_v, out_hbm.at[pl.ds(base, BPW)])


@jax.jit
def _bpr_sc(u, i, j, w1, h1):
    mesh = plsc.VectorSubcoreMesh(core_axis_name="c", subcore_axis_name="s",
                                  num_cores=NC, num_subcores=NS)
    return pl.kernel(
        _bpr_body,
        out_type=jax.ShapeDtypeStruct((B,), jnp.float32),
        mesh=mesh,
        scratch_types=[
            pltpu.VMEM((BPW,), jnp.int32),
            pltpu.VMEM((BPW,), jnp.int32),
            pltpu.VMEM((BPW,), jnp.int32),
            pltpu.VMEM((D * BPW,), jnp.int32),
            pltpu.VMEM((D * BPW,), jnp.int32),
            pltpu.VMEM((D * BPW,), jnp.int32),
            pltpu.VMEM((D * BPW,), jnp.float32),
            pltpu.VMEM((D * BPW,), jnp.float32),
            pltpu.VMEM((D * BPW,), jnp.float32),
            pltpu.VMEM((BPW,), jnp.float32),
            pltpu.SemaphoreType.DMA,
        ],
        compiler_params=pltpu.CompilerParams(needs_layout_passes=False),
    )(u, i, j, w1, h1)


def kernel(u, i, j, W, H):
    return _bpr_sc(u.astype(jnp.int32), i.astype(jnp.int32),
                   j.astype(jnp.int32), W.T.reshape(-1), H.T.reshape(-1))


# speedup vs baseline: 8.2527x; 8.2527x over previous
import jax
import jax.numpy as jnp
from jax import lax
from jax.experimental import pallas as pl
from jax.experimental.pallas import tpu as pltpu
from jax.experimental.pallas import tpu_sc as plsc

B = 16384
D = 32
WD = 1e-05
NC = 2
NS = 16
NW = NC * NS
BPW = B // NW
HALF = BPW // 2
L = 16
K = 128


def _bpr_body(u_hbm, i_hbm, j_hbm, w_hbm, h_hbm, out_hbm,
              u_v, i_v, j_v, ue2, ie2, je2, tbuf, out_v, sem):
    wid = lax.axis_index("s") * NC + lax.axis_index("c")
    base = wid * BPW

    pltpu.sync_copy(u_hbm.at[pl.ds(base, BPW)], u_v)
    pltpu.sync_copy(i_hbm.at[pl.ds(base, BPW)], i_v)
    pltpu.sync_copy(j_hbm.at[pl.ds(base, BPW)], j_v)

    third = jnp.float32(1.0 / 3.0)
    fifth = jnp.float32(1.0 / 5.0)
    seventh = jnp.float32(1.0 / 7.0)
    zero = jnp.zeros((L,), jnp.float32)
    lanes = lax.iota(jnp.int32, L)

    def row_term(rloc):
        t = zero
        for h in range(D // L):
            ue = ue2[rloc, pl.ds(h * L, L)]
            ie = ie2[rloc, pl.ds(h * L, L)]
            je = je2[rloc, pl.ds(h * L, L)]
            x = ue * (ie - je)
            e = jnp.exp(-jnp.abs(x))
            s = e / (e + 2.0)
            s2 = s * s
            log1p = (2.0 * s) * (1.0 + s2 * (third + s2 * (fifth + s2 * seventh)))
            logsig = jnp.minimum(x, zero) - log1p
            sq = ue * ue + ie * ie + je * je
            t = t + (jnp.float32(WD / D) * sq - jnp.float32(1.0 / D) * logsig)
        return t

    for half in range(2):
        roff = half * HALF

        def fire_group(g, _):
            rl0 = g * L
            uv = u_v[pl.ds(roff + rl0, L)]
            iv = i_v[pl.ds(roff + rl0, L)]
            jv = j_v[pl.ds(roff + rl0, L)]
            for k in range(L):
                dst = pl.ds(rl0 + k, 1)
                pltpu.async_copy(w_hbm.at[pl.ds(uv[k], 1), :], ue2.at[dst, :], sem)
                pltpu.async_copy(h_hbm.at[pl.ds(iv[k], 1), :], ie2.at[dst, :], sem)
                pltpu.async_copy(h_hbm.at[pl.ds(jv[k], 1), :], je2.at[dst, :], sem)
            return _

        def drain(r, _):
            d0 = pl.ds(0, 1)
            pltpu.make_async_copy(w_hbm.at[d0, :], ue2.at[d0, :], sem).wait()
            pltpu.make_async_copy(h_hbm.at[d0, :], ie2.at[d0, :], sem).wait()
            pltpu.make_async_copy(h_hbm.at[d0, :], je2.at[d0, :], sem).wait()
            return _

        def gather_chunk(c, _):
            g0 = c * (K // L)
            lax.fori_loop(g0, g0 + K // L, fire_group, None)
            lax.fori_loop(0, K, drain, None)
            return _

        lax.fori_loop(0, HALF // K, gather_chunk, None)

        def row_body(g, _):
            rl0 = g * 4
            for k in range(4):
                rloc = rl0 + k
                plsc.store_scatter(tbuf, [lanes * BPW + (roff + rloc)],
                                   row_term(rloc))
            return _

        lax.fori_loop(0, HALF // 4, row_body, None)

    def sum_body(c, _):
        r0 = c * L
        acc = tbuf[pl.ds(r0, L)]
        for l in range(1, L):
            acc = acc + tbuf[pl.ds(l * BPW + r0, L)]
        out_v[pl.ds(r0, L)] = acc
        return _

    lax.fori_loop(0, BPW // L, sum_body, None)
    pltpu.sync_copy(out_v, out_hbm.at[pl.ds(base, BPW)])


@jax.jit
def _bpr_sc(u, i, j, w, h):
    mesh = plsc.VectorSubcoreMesh(core_axis_name="c", subcore_axis_name="s",
                                  num_cores=NC, num_subcores=NS)
    return pl.kernel(
        _bpr_body,
        out_type=jax.ShapeDtypeStruct((B,), jnp.float32),
        mesh=mesh,
        scratch_types=[
            pltpu.VMEM((BPW,), jnp.int32),
            pltpu.VMEM((BPW,), jnp.int32),
            pltpu.VMEM((BPW,), jnp.int32),
            pltpu.VMEM((HALF, D), jnp.float32),
            pltpu.VMEM((HALF, D), jnp.float32),
            pltpu.VMEM((HALF, D), jnp.float32),
            pltpu.VMEM((L * BPW,), jnp.float32),
            pltpu.VMEM((BPW,), jnp.float32),
            pltpu.SemaphoreType.DMA,
        ],
        compiler_params=pltpu.CompilerParams(needs_layout_passes=False),
    )(u, i, j, w, h)


def kernel(u, i, j, W, H):
    return _bpr_sc(u.astype(jnp.int32), i.astype(jnp.int32),
                   j.astype(jnp.int32), W, H)
